# baseline (speedup 1.0000x reference)
import functools
import jax
import jax.numpy as jnp
from jax import lax
from jax.experimental import pallas as pl
from jax.experimental.pallas import tpu as pltpu
from jax.experimental.pallas import tpu_sc as plsc

_N = 35
_E = 1225
_V = 6
_EP = 1536
_EW = 96
_NP = 48
_HP = 128
_LAYERS = ((1, 36, 48), (36, 24, 32), (24, 5, 16))

_DN = lax.GatherDimensionNumbers(
    offset_dims=(), collapsed_slice_dims=(0,), start_index_map=(0,))


def _spl(vec, lane):
    idx = jnp.full((16, 1), lane, jnp.int32)
    return lax.gather(vec, idx, _DN, (1,),
                      mode=lax.GatherScatterMode.PROMISE_IN_BOUNDS)


def _rb(v):
    return v.astype(jnp.bfloat16).astype(jnp.float32)


def _sc_call(h0, ea, src, dst, w1, b1, r1, c1, w2, b2, r2, c2, w3, b3, r3, c3):
    mesh = plsc.VectorSubcoreMesh(core_axis_name="c", subcore_axis_name="s")
    f32 = jnp.float32

    @functools.partial(
        pl.kernel, mesh=mesh,
        out_type=jax.ShapeDtypeStruct((_N * _NP,), f32),
        scratch_types=[
            pltpu.VMEM((_EW, 16), f32),
            pltpu.VMEM((_EW,), jnp.int32),
            pltpu.VMEM((_EW,), jnp.int32),
            pltpu.VMEM((_EW, _HP), f32),
            pltpu.VMEM((_EW, 48), f32),
            pltpu.VMEM((_NP, _HP), f32),
            pltpu.VMEM((_NP, _HP), f32),
            pltpu.VMEM((16, _HP), f32),
            pltpu.VMEM((_EW,), jnp.int32),
            pltpu.VMEM((16,), jnp.int32),
            pltpu.VMEM((48,), f32),
            pltpu.VMEM((48,), jnp.int32),
            pltpu.VMEM((6, 48), f32),
            pltpu.VMEM((6, 1152), f32),
            pltpu.VMEM((6, 384), f32),
            pltpu.VMEM((48,), f32),
            pltpu.VMEM((1152,), f32),
            pltpu.VMEM((384,), f32),
            pltpu.VMEM((1, 48), f32),
            pltpu.VMEM((36, 32), f32),
            pltpu.VMEM((24, 16), f32),
            pltpu.VMEM((48,), f32),
            pltpu.VMEM((32,), f32),
            pltpu.VMEM((16,), f32),
            pltpu.VMEM_SHARED((_NP, _HP), f32),
            pltpu.HBM((2 * _NP, _HP), f32),
            pltpu.SemaphoreType.DMA,
        ],
    )
    def k(h0_hbm, ea_hbm, src_hbm, dst_hbm,
          w1_hbm, b1_hbm, r1_hbm, c1_hbm,
          w2_hbm, b2_hbm, r2_hbm, c2_hbm,
          w3_hbm, b3_hbm, r3_hbm, c3_hbm,
          out_hbm,
          ea_v, src_v, dst_v, xr_v, msg_v, acc_v, h_v,
          nh_v, src2_v, own3_v, rv_v, idx48,
          w1_v, w2_v, w3_v, b1_v, b2_v, b3_v,
          r1_v, r2_v, r3_v, c1_v, c2_v, c3_v,
          sh_acc, hsc, sem):
        cid = lax.axis_index("c")
        s = lax.axis_index("s")
        zero16 = jnp.zeros((16,), f32)
        base = s * _EW
        n0 = s * 3

        pltpu.sync_copy(ea_hbm.at[pl.ds(base, _EW)], ea_v)
        pltpu.sync_copy(src_hbm.at[pl.ds(base, _EW)], src_v)
        pltpu.sync_copy(dst_hbm.at[pl.ds(base, _EW)], dst_v)
        pltpu.sync_copy(w1_hbm, w1_v)
        pltpu.sync_copy(w2_hbm, w2_v)
        pltpu.sync_copy(w3_hbm, w3_v)
        pltpu.sync_copy(b1_hbm, b1_v)
        pltpu.sync_copy(b2_hbm, b2_v)
        pltpu.sync_copy(b3_hbm, b3_v)
        pltpu.sync_copy(r1_hbm, r1_v)
        pltpu.sync_copy(r2_hbm, r2_v)
        pltpu.sync_copy(r3_hbm, r3_v)
        pltpu.sync_copy(c1_hbm, c1_v)
        pltpu.sync_copy(c2_hbm, c2_v)
        pltpu.sync_copy(c3_hbm, c3_v)

        iota16 = lax.iota(jnp.int32, 16)
        for g in range(3):
            idx48[pl.ds(g * 16, 16)] = iota16 + (g * 16)
        for g in range(_EW // 16):
            sl = pl.ds(g * 16, 16)
            src2_v[sl] = src_v[sl] + cid * _NP
        own3_v[pl.ds(0, 16)] = jnp.where(
            iota16 < 3, cid * _NP + n0 + iota16,
            cid * _NP + 36 + ((n0 + iota16) % 12))
        for r in range(3, 16):
            for j in range(_HP // 16):
                nh_v[r, pl.ds(j * 16, 16)] = zero16

        one16 = jnp.full((16,), 1.0, f32)

        wrefs = ((w1_v, b1_v, r1_v, c1_v), (w2_v, b2_v, r2_v, c2_v),
                 (w3_v, b3_v, r3_v, c3_v))

        for li, (ic, oc, ocp) in enumerate(_LAYERS):
            w_v, b_v, r_v, c_v = wrefs[li]
            jn = ocp // 16
            rnd = li > 0

            def zbody(e, carry):
                for j in range(jn):
                    msg_v[e, pl.ds(j * 16, 16)] = zero16
                return carry
            lax.fori_loop(0, _EW, zbody, 0)
            for r in range(_NP):
                for j in range(_HP // 16):
                    acc_v[r, pl.ds(j * 16, 16)] = zero16

            @pl.when(s == 0)
            def _():
                pltpu.sync_copy(acc_v, sh_acc)
            plsc.subcore_barrier()

            if li == 0:
                pltpu.async_copy(h0_hbm.at[src_v], xr_v, sem).wait()
                pltpu.sync_copy(h0_hbm, h_v)
            else:
                pltpu.async_copy(hsc.at[src2_v], xr_v, sem).wait()
                pltpu.sync_copy(hsc.at[pl.ds(cid * _NP, _NP)], h_v)

            IB = 3
            for ib in range(0, ic, IB):
                ii = list(range(ib, min(ib + IB, ic)))
                wv = {}
                bv = {}
                for i in ii:
                    for j in range(jn):
                        off = i * ocp + j * 16
                        bv[(i, j)] = b_v[pl.ds(off, 16)]
                        for v in range(_V):
                            wv[(i, j, v)] = w_v[v, pl.ds(off, 16)]
                blocks = sorted({i // 16 for i in ii})

                def ebody(e, carry):
                    ear = ea_v[e, pl.ds(0, 16)]
                    asp = [_spl(ear, v) for v in range(_V)]
                    xrow = {blk: xr_v[e, pl.ds(blk * 16, 16)]
                            for blk in blocks}
                    for i in ii:
                        xsp = _spl(xrow[i // 16], i % 16)
                        if rnd:
                            xsp = _rb(xsp)
                        for j in range(jn):
                            filt = bv[(i, j)]
                            for v in range(_V):
                                filt = filt + asp[v] * wv[(i, j, v)]
                            filt = jnp.maximum(filt, 0.0)
                            if rnd:
                                filt = _rb(filt)
                            sl = pl.ds(j * 16, 16)
                            msg_v[e, sl] = msg_v[e, sl] + xsp * filt
                    return carry

                lax.fori_loop(0, _EW, ebody, 0)

            def abody(g, carry):
                dstg = dst_v[pl.ds(g * 16, 16)]
                for lane in range(16):
                    row = dstg[lane]
                    e = g * 16 + lane
                    for j in range(jn):
                        sl = pl.ds(j * 16, 16)
                        acc_v[row, sl] = acc_v[row, sl] + msg_v[e, sl]
                    cs = pl.ds(112, 16)
                    acc_v[row, cs] = acc_v[row, cs] + one16
                return carry

            lax.fori_loop(0, _EW // 16, abody, 0)

            pltpu.sync_copy(acc_v, sh_acc.at[idx48], add=True)
            plsc.subcore_barrier()

            pltpu.sync_copy(sh_acc, acc_v)
            nb = (ic + 15) // 16
            for nl in range(3):
                n = n0 + nl
                cr = acc_v[n, pl.ds(112, 16)]
                inv = 1.0 / jnp.maximum(cr, 1.0)
                hrow = [h_v[n, pl.ds(blk * 16, 16)] for blk in range(nb)]
                rsum = [c_v[pl.ds(j * 16, 16)] for j in range(jn)]
                for i in range(ic):
                    hs = _rb(_spl(hrow[i // 16], i % 16))
                    for j in range(jn):
                        rsum[j] = rsum[j] + hs * r_v[i, pl.ds(j * 16, 16)]
                for j in range(jn):
                    sl = pl.ds(j * 16, 16)
                    agg = acc_v[n, sl] * inv
                    nh_v[nl, sl] = jnp.maximum(agg + rsum[j], 0.0)
                for j in range(jn, _HP // 16):
                    nh_v[nl, pl.ds(j * 16, 16)] = zero16
            pltpu.sync_copy(nh_v, hsc.at[own3_v])
            plsc.subcore_barrier()

        pltpu.sync_copy(hsc.at[pl.ds(cid * _NP, _NP)], h_v)
        for al in range(3):
            a = n0 + al

            @pl.when((a < _N) & (cid == 0))
            def _():
                va = h_v[a, pl.ds(0, 16)]
                perms = [jnp.bitwise_xor(iota16, sh).reshape(16, 1)
                         for sh in (8, 4, 2, 1)]
                for g in range(3):
                    def bbody(b16, racc):
                        vb = h_v[g * 16 + b16, pl.ds(0, 16)]
                        d = jnp.abs(va - vb)
                        for pm in perms:
                            d = d + lax.gather(
                                d, pm, _DN, (1,),
                                mode=lax.GatherScatterMode.PROMISE_IN_BOUNDS)
                        m = iota16 == b16
                        return jnp.where(m, d, racc)
                    racc = lax.fori_loop(0, 16, bbody, zero16)
                    rv_v[pl.ds(g * 16, 16)] = racc
                pltpu.sync_copy(rv_v, out_hbm.at[pl.ds(a * 48, 48)])

    return k(h0, ea, src, dst, w1, b1, r1, c1, w2, b2, r2, c2,
             w3, b3, r3, c3)


def _pad_w(W, b, root, bias, ic, oc, ocp):
    W3 = W.reshape(_V, ic, oc)
    Wp = jnp.zeros((_V, ic, ocp), jnp.float32).at[:, :, :oc].set(W3)
    bp = jnp.zeros((ic, ocp), jnp.float32).at[:, :oc].set(b.reshape(ic, oc))
    rp = jnp.zeros((ic, ocp), jnp.float32).at[:, :oc].set(root)
    cp = jnp.zeros((ocp,), jnp.float32).at[:oc].set(bias)
    return Wp.reshape(_V, ic * ocp), bp.reshape(ic * ocp), rp, cp


def kernel(x, edge_attr, edge_index,
           W1, b1, root1, bias1,
           W2, b2, root2, bias2,
           W3, b3, root3, bias3):
    f32 = jnp.float32
    rbf = lambda t: t.astype(jnp.bfloat16).astype(f32)
    h0 = jnp.zeros((_NP, _HP), f32).at[:_N, 0].set(x[:, 0])
    ea = jnp.zeros((_EP, 16), f32).at[:_E, :_V].set(rbf(edge_attr))
    src = jnp.zeros((_EP,), jnp.int32).at[:_E].set(edge_index[0])
    dst = jnp.full((_EP,), _N, jnp.int32).at[:_E].set(edge_index[1])
    w1, b1p, r1, c1 = _pad_w(rbf(W1), b1, root1, bias1, *_LAYERS[0])
    w2, b2p, r2, c2 = _pad_w(rbf(W2), b2, root2, bias2, *_LAYERS[1])
    w3, b3p, r3, c3 = _pad_w(rbf(W3), b3, root3, bias3, *_LAYERS[2])
    r1, r2, r3 = rbf(r1), rbf(r2), rbf(r3)
    res = _sc_call(h0, ea, src, dst, w1, b1p, r1, c1,
                   w2, b2p, r2, c2, w3, b3p, r3, c3)
    return res.reshape(_N, _NP)[:, :_N]


# speedup vs baseline: 12.0514x; 12.0514x over previous
import jax
import jax.numpy as jnp
from jax.experimental import pallas as pl

_N = 35
_E = 1225
_V = 6
_LAYERS = ((1, 36), (36, 24), (24, 5))


def _mgn_body(x_ref, ea_ref, ei_ref,
              W1_ref, b1_ref, r1_ref, c1_ref,
              W2_ref, b2_ref, r2_ref, c2_ref,
              W3_ref, b3_ref, r3_ref, c3_ref,
              out_ref):
    f32 = jnp.float32
    i32 = jnp.int32

    def dot(a, b):
        return jax.lax.dot_general(a, b, (((1,), (0,)), ((), ())),
                                   preferred_element_type=f32)

    def dotT(a, b):
        return jax.lax.dot_general(a, b, (((0,), (0,)), ((), ())),
                                   preferred_element_type=f32)

    def rb(t):
        return t.astype(jnp.bfloat16).astype(f32)

    def split(t):
        hi = rb(t)
        return hi, rb(t - hi)

    def dot_xx(a, b):
        a_hi, a_lo = split(a)
        b_hi, b_lo = split(b)
        return ((dot(a_hi, b_hi) + dot(a_hi, b_lo))
                + (dot(a_lo, b_hi) + dot(a_lo, b_lo)))

    ei = ei_ref[...]
    src = ei[0:1, :]
    dst = ei[1:2, :]
    ea = ea_ref[...]

    src_ohT = (src == jax.lax.broadcasted_iota(i32, (_N, _E), 0)).astype(f32)
    dst_ohT = (dst == jax.lax.broadcasted_iota(i32, (_N, _E), 0)).astype(f32)
    cnt = jnp.sum(dst_ohT, axis=1, keepdims=True)
    inv = 1.0 / jnp.maximum(cnt, 1.0)

    h = x_ref[...]
    params = ((W1_ref, b1_ref, r1_ref, c1_ref),
              (W2_ref, b2_ref, r2_ref, c2_ref),
              (W3_ref, b3_ref, r3_ref, c3_ref))
    for (ic, oc), (W_ref, b_ref, r_ref, c_ref) in zip(_LAYERS, params):
        filt = jnp.maximum(dot(ea, W_ref[...]) + b_ref[...][None, :], 0.0)
        h_hi, h_lo = split(h)
        xj = dotT(src_ohT, h_hi) + dotT(src_ohT, h_lo)
        h_hi2, h_lo2 = split(xj)
        if ic == 1:
            msg = xj * filt
        else:
            qr = jax.lax.broadcasted_iota(i32, (ic, ic * oc), 0)
            qc = jax.lax.broadcasted_iota(i32, (ic, ic * oc), 1)
            Q = (qc // oc == qr).astype(f32)
            rr = jax.lax.broadcasted_iota(i32, (ic * oc, oc), 0)
            rc = jax.lax.broadcasted_iota(i32, (ic * oc, oc), 1)
            R = (rr % oc == rc).astype(f32)
            p = (dot(h_hi2, Q) + dot(h_lo2, Q)) * filt
            p_hi, p_lo = split(p)
            msg = dot(p_hi, R) + dot(p_lo, R)
        m_hi, m_lo = split(msg)
        agg = (dot(dst_ohT, m_hi) + dot(dst_ohT, m_lo)) * inv
        h = jnp.maximum(agg + dot_xx(h, r_ref[...])
                        + c_ref[...][None, :], 0.0)

    eye = (jax.lax.broadcasted_iota(i32, (_N, _N), 0)
           == jax.lax.broadcasted_iota(i32, (_N, _N), 1)).astype(f32)
    h_hi, h_lo = split(h)
    hT = dotT(h_hi, eye) + dotT(h_lo, eye)
    acc = jnp.zeros((_N, _N), f32)
    for k in range(_LAYERS[-1][1]):
        acc = acc + jnp.abs(hT[k:k + 1, :] - h[:, k:k + 1])
    out_ref[...] = acc


def kernel(x, edge_attr, edge_index,
           W1, b1, root1, bias1,
           W2, b2, root2, bias2,
           W3, b3, root3, bias3):
    return pl.pallas_call(
        _mgn_body,
        out_shape=jax.ShapeDtypeStruct((_N, _N), jnp.float32),
    )(x, edge_attr, edge_index,
      W1, b1, root1, bias1,
      W2, b2, root2, bias2,
      W3, b3, root3, bias3)
